# baseline (speedup 1.0000x reference)
import functools

import jax
import jax.numpy as jnp
from jax import lax
from jax.experimental import pallas as pl
from jax.experimental.pallas import tpu as pltpu
from jax.experimental.pallas import tpu_sc as plsc

TOP_K = 8
NUM_EXPERTS = 64
ROW_BLOCK = 1024

NEG_INF = float("-inf")


def _logits_t_kernel(t_per_w, h_ref, w_ref, b_ref, o_ref):
    x = h_ref[:, :]
    w = w_ref[:, :]
    logits_t = jax.lax.dot_general(
        w, x, (((0,), (1,)), ((), ())),
        preferred_element_type=jnp.float32) + b_ref[:, :]
    for w_loc in range(ROW_BLOCK // t_per_w):
        o_ref[w_loc] = logits_t[:, w_loc * t_per_w:(w_loc + 1) * t_per_w]


def _make_logits_t(hidden_states, weight, bias, nw):
    n_tokens, hidden = hidden_states.shape
    t_per_w = n_tokens // nw
    w_per_block = ROW_BLOCK // t_per_w
    grid = (n_tokens // ROW_BLOCK,)
    bias2 = bias.reshape(NUM_EXPERTS, 1)
    return pl.pallas_call(
        functools.partial(_logits_t_kernel, t_per_w),
        grid=grid,
        in_specs=[
            pl.BlockSpec((ROW_BLOCK, hidden), lambda i: (i, 0)),
            pl.BlockSpec((hidden, NUM_EXPERTS), lambda i: (0, 0)),
            pl.BlockSpec((NUM_EXPERTS, 1), lambda i: (0, 0)),
        ],
        out_specs=pl.BlockSpec(
            (w_per_block, NUM_EXPERTS, t_per_w), lambda i: (i, 0, 0)),
        out_shape=jax.ShapeDtypeStruct(
            (nw, NUM_EXPERTS, t_per_w), jnp.float32),
    )(hidden_states, weight, bias2)


def _make_router(n_tokens):
    info = plsc.get_sparse_core_info()
    nc, ns, nl = info.num_cores, info.num_subcores, info.num_lanes
    nw = nc * ns
    t_per_w = n_tokens // nw
    n_groups = t_per_w // nl

    mesh = plsc.VectorSubcoreMesh(core_axis_name="c", subcore_axis_name="s")

    @functools.partial(
        pl.kernel, mesh=mesh,
        out_type=[
            jax.ShapeDtypeStruct((TOP_K, n_tokens), jnp.float32),
            jax.ShapeDtypeStruct((TOP_K, n_tokens), jnp.int32),
        ],
        scratch_types=[
            pltpu.VMEM((NUM_EXPERTS, t_per_w), jnp.float32),
            pltpu.VMEM((TOP_K, t_per_w), jnp.float32),
            pltpu.VMEM((TOP_K, t_per_w), jnp.int32),
        ],
    )
    def router(lt_hbm, ow_hbm, oi_hbm, chunk, ovw, oiw):
        wid = lax.axis_index("s") * nc + lax.axis_index("c")
        base = wid * t_per_w
        pltpu.sync_copy(lt_hbm.at[wid], chunk)

        @plsc.parallel_loop(0, n_groups, step=1, unroll=2)
        def group_body(g):
            col = g * nl
            m = [jnp.full((nl,), NEG_INF, jnp.float32) for _ in range(TOP_K)]
            ix = [jnp.zeros((nl,), jnp.int32) for _ in range(TOP_K)]
            for e in range(NUM_EXPERTS):
                cv = chunk[e, pl.ds(col, nl)]
                ci = jnp.full((nl,), e, jnp.int32)
                for k in range(min(e + 1, TOP_K)):
                    gt = cv > m[k]
                    nm = jnp.where(gt, cv, m[k])
                    cv = jnp.where(gt, m[k], cv)
                    ni = jnp.where(gt, ci, ix[k])
                    ci = jnp.where(gt, ix[k], ci)
                    m[k] = nm
                    ix[k] = ni
            es = [jnp.exp(v - m[0]) for v in m]
            s = es[0]
            for t in es[1:]:
                s = s + t
            inv = 1.0 / s
            for k in range(TOP_K):
                ovw[k, pl.ds(col, nl)] = es[k] * inv
                oiw[k, pl.ds(col, nl)] = ix[k]

        pltpu.sync_copy(ovw, ow_hbm.at[:, pl.ds(base, t_per_w)])
        pltpu.sync_copy(oiw, oi_hbm.at[:, pl.ds(base, t_per_w)])

    return router


@jax.jit
def kernel(hidden_states, weight, bias):
    n_tokens = hidden_states.shape[0]
    info = plsc.get_sparse_core_info()
    nw = info.num_cores * info.num_subcores
    logits_t = _make_logits_t(hidden_states, weight, bias, nw)
    router = _make_router(n_tokens)
    ow, oi = router(logits_t)
    return ow.T, oi.T


# speedup vs baseline: 1.2091x; 1.2091x over previous
import functools

import jax
import jax.numpy as jnp
from jax import lax
from jax.experimental import pallas as pl
from jax.experimental.pallas import tpu as pltpu
from jax.experimental.pallas import tpu_sc as plsc

TOP_K = 8
NUM_EXPERTS = 64
ROW_BLOCK = 1024

NEG_INF = float("-inf")


def _logits_t_kernel(t_per_w, h_ref, w_ref, b_ref, o_ref):
    x = h_ref[:, :]
    w = w_ref[:, :]
    logits_t = jax.lax.dot_general(
        w, x, (((0,), (1,)), ((), ())),
        preferred_element_type=jnp.float32) + b_ref[:, :]
    for w_loc in range(ROW_BLOCK // t_per_w):
        o_ref[w_loc] = logits_t[:, w_loc * t_per_w:(w_loc + 1) * t_per_w]


def _make_logits_t(hidden_states, weight, bias, nw):
    n_tokens, hidden = hidden_states.shape
    t_per_w = n_tokens // nw
    w_per_block = ROW_BLOCK // t_per_w
    grid = (n_tokens // ROW_BLOCK,)
    bias2 = bias.reshape(NUM_EXPERTS, 1)
    return pl.pallas_call(
        functools.partial(_logits_t_kernel, t_per_w),
        grid=grid,
        in_specs=[
            pl.BlockSpec((ROW_BLOCK, hidden), lambda i: (i, 0)),
            pl.BlockSpec((hidden, NUM_EXPERTS), lambda i: (0, 0)),
            pl.BlockSpec((NUM_EXPERTS, 1), lambda i: (0, 0)),
        ],
        out_specs=pl.BlockSpec(
            (w_per_block, NUM_EXPERTS, t_per_w), lambda i: (i, 0, 0)),
        out_shape=jax.ShapeDtypeStruct(
            (nw, NUM_EXPERTS, t_per_w), jnp.float32),
    )(hidden_states, weight, bias2)


def _make_router(n_tokens):
    info = plsc.get_sparse_core_info()
    nc, ns, nl = info.num_cores, info.num_subcores, info.num_lanes
    nw = nc * ns
    t_per_w = n_tokens // nw
    n_groups = t_per_w // nl

    mesh = plsc.VectorSubcoreMesh(core_axis_name="c", subcore_axis_name="s")

    @functools.partial(
        pl.kernel, mesh=mesh,
        out_type=[
            jax.ShapeDtypeStruct((TOP_K, n_tokens), jnp.float32),
            jax.ShapeDtypeStruct((TOP_K, n_tokens), jnp.int32),
        ],
        scratch_types=[
            pltpu.VMEM((NUM_EXPERTS, t_per_w), jnp.float32),
            pltpu.VMEM((TOP_K, t_per_w), jnp.float32),
            pltpu.VMEM((TOP_K, t_per_w), jnp.int32),
        ],
    )
    def router(lt_hbm, ow_hbm, oi_hbm, chunk, ovw, oiw):
        wid = lax.axis_index("s") * nc + lax.axis_index("c")
        base = wid * t_per_w
        pltpu.sync_copy(lt_hbm.at[wid], chunk)

        @plsc.parallel_loop(0, n_groups, step=1)
        def group_body(g):
            col = g * nl
            m = [jnp.full((nl,), NEG_INF, jnp.float32) for _ in range(TOP_K)]
            ix = [jnp.zeros((nl,), jnp.int32) for _ in range(TOP_K)]
            for e in range(NUM_EXPERTS):
                v = chunk[e, pl.ds(col, nl)]
                vi = jnp.full((nl,), e, jnp.int32)
                kmax = min(e + 1, TOP_K)
                gts = [v > m[k] for k in range(kmax)]
                new_m = []
                new_ix = []
                for k in range(kmax):
                    vk = jnp.where(gts[k], v, m[k])
                    ik = jnp.where(gts[k], vi, ix[k])
                    if k > 0:
                        vk = jnp.where(gts[k - 1], m[k - 1], vk)
                        ik = jnp.where(gts[k - 1], ix[k - 1], ik)
                    new_m.append(vk)
                    new_ix.append(ik)
                m[:kmax] = new_m
                ix[:kmax] = new_ix
            es = [jnp.exp(v - m[0]) for v in m]
            s = es[0]
            for t in es[1:]:
                s = s + t
            inv = 1.0 / s
            for k in range(TOP_K):
                ovw[k, pl.ds(col, nl)] = es[k] * inv
                oiw[k, pl.ds(col, nl)] = ix[k]

        pltpu.sync_copy(ovw, ow_hbm.at[:, pl.ds(base, t_per_w)])
        pltpu.sync_copy(oiw, oi_hbm.at[:, pl.ds(base, t_per_w)])

    return router


@jax.jit
def kernel(hidden_states, weight, bias):
    n_tokens = hidden_states.shape[0]
    info = plsc.get_sparse_core_info()
    nw = info.num_cores * info.num_subcores
    logits_t = _make_logits_t(hidden_states, weight, bias, nw)
    router = _make_router(n_tokens)
    ow, oi = router(logits_t)
    return ow.T, oi.T
